# Initial kernel scaffold; baseline (speedup 1.0000x reference)
import jax
import jax.numpy as jnp
from jax.experimental import pallas as pl


def kernel(x, edge_index, batch, patient_features, W1, b1, W2, b2, W3, b3, Wp1, bp1, Wp2, bp2, Wc1, bc1, Wc2, bc2):
    raise NotImplementedError("write your pallas kernel here")


# speedup vs baseline: 8.4479x; 8.4479x over previous
import functools

import jax
import jax.numpy as jnp
from jax import lax
from jax.experimental import pallas as pl
from jax.experimental.pallas import tpu as pltpu
from jax.experimental.pallas import tpu_sc as plsc

N = 10000
E = 320000
D = 128
H = 128
G = 256
PD = 3
PH = 64
OUT = 1500

NC = 2
NS = 16
NW = NC * NS
NPAD = 10240
RPT = NPAD // NS
EPT_REAL = E // NW
CHUNKS = 80
EPT = CHUNKS * 128
GRP = 8
NGRP = CHUNKS // GRP
SINK = N
BR = 256
OUTP = 1536

f32 = jnp.float32
_PREC = lax.Precision.HIGHEST

_MESH = plsc.VectorSubcoreMesh(core_axis_name="c", subcore_axis_name="s")


@functools.partial(
    pl.kernel,
    out_type=jax.ShapeDtypeStruct((NC, NPAD), f32),
    mesh=_MESH,
    scratch_types=[
        pltpu.VMEM((CHUNKS, 128), jnp.int32),
        pltpu.VMEM((128,), f32),
        pltpu.VMEM_SHARED((NPAD,), f32),
    ],
)
def _sc_degree(dst_hbm, ones_hbm, zeros_hbm, out_hbm, dst_v, ones_v, acc):
    cid = lax.axis_index("c")
    sid = lax.axis_index("s")
    wid = sid * NC + cid
    pltpu.sync_copy(dst_hbm.at[wid], dst_v)
    pltpu.sync_copy(ones_hbm, ones_v)
    pltpu.sync_copy(zeros_hbm, acc.at[pl.ds(sid * RPT, RPT)])
    plsc.subcore_barrier()

    def body(c, carry):
        pltpu.sync_copy(ones_v, acc.at[dst_v.at[c]], add=True)
        return carry

    lax.fori_loop(0, CHUNKS, body, 0)
    plsc.subcore_barrier()
    pltpu.sync_copy(acc.at[pl.ds(sid * RPT, RPT)],
                    out_hbm.at[cid, pl.ds(sid * RPT, RPT)])


@functools.partial(
    pl.kernel,
    out_type=jax.ShapeDtypeStruct((NC, NPAD, H), f32),
    mesh=_MESH,
    scratch_types=[
        pltpu.VMEM((GRP, 128), jnp.int32),
        pltpu.VMEM((GRP, 128), jnp.int32),
        pltpu.VMEM((2, 128, H), f32),
        pltpu.VMEM_SHARED((NPAD, H), f32),
        pltpu.SemaphoreType.DMA,
        pltpu.SemaphoreType.DMA,
    ],
)
def _sc_edge_agg(u_hbm, src_hbm, dst_hbm, zeros_hbm, out_hbm,
                 src_v, dst_v, rows_v, acc, sem0, sem1):
    cid = lax.axis_index("c")
    sid = lax.axis_index("s")
    wid = sid * NC + cid
    pltpu.sync_copy(zeros_hbm, acc.at[pl.ds(sid * RPT, RPT)])
    plsc.subcore_barrier()

    def group(g, carry):
        pltpu.sync_copy(src_hbm.at[wid, pl.ds(g * GRP, GRP)], src_v)
        pltpu.sync_copy(dst_hbm.at[wid, pl.ds(g * GRP, GRP)], dst_v)
        pltpu.async_copy(u_hbm.at[src_v.at[0]], rows_v.at[0], sem0)

        def pair(i, c2):
            ca = 2 * i
            cb = ca + 1
            db = pltpu.async_copy(u_hbm.at[src_v.at[cb]], rows_v.at[1], sem1)
            pltpu.make_async_copy(u_hbm.at[src_v.at[ca]], rows_v.at[0],
                                  sem0).wait()
            pltpu.sync_copy(rows_v.at[0], acc.at[dst_v.at[ca]], add=True)

            @pl.when(i < GRP // 2 - 1)
            def _():
                pltpu.async_copy(u_hbm.at[src_v.at[ca + 2]], rows_v.at[0],
                                 sem0)

            db.wait()
            pltpu.sync_copy(rows_v.at[1], acc.at[dst_v.at[cb]], add=True)
            return c2

        lax.fori_loop(0, GRP // 2, pair, 0)
        return carry

    lax.fori_loop(0, NGRP, group, 0)
    plsc.subcore_barrier()
    pltpu.sync_copy(acc.at[pl.ds(sid * RPT, RPT)],
                    out_hbm.at[cid, pl.ds(sid * RPT, RPT)])


def _tc_mm1_body(x_ref, w_ref, d0_ref, d1_ref, u_ref, dinv_ref):
    dinv = lax.rsqrt(1.0 + d0_ref[...] + d1_ref[...])
    dinv_ref[...] = dinv
    u_ref[...] = dinv * jnp.dot(x_ref[...], w_ref[...],
                                preferred_element_type=f32, precision=_PREC)


def _tc_mm1(x_pad, W1, d0, d1):
    return pl.pallas_call(
        _tc_mm1_body,
        grid=(NPAD // BR,),
        in_specs=[
            pl.BlockSpec((BR, D), lambda i: (i, 0)),
            pl.BlockSpec((D, H), lambda i: (0, 0)),
            pl.BlockSpec((BR, 1), lambda i: (i, 0)),
            pl.BlockSpec((BR, 1), lambda i: (i, 0)),
        ],
        out_specs=[
            pl.BlockSpec((BR, H), lambda i: (i, 0)),
            pl.BlockSpec((BR, 1), lambda i: (i, 0)),
        ],
        out_shape=[
            jax.ShapeDtypeStruct((NPAD, H), f32),
            jax.ShapeDtypeStruct((NPAD, 1), f32),
        ],
    )(x_pad, W1, d0, d1)


def _tc_layer_body(s_ref, u_ref, dinv_ref, b_ref, w_ref, o_ref):
    dinv = dinv_ref[...]
    a = jnp.maximum(dinv * (s_ref[0] + s_ref[1] + u_ref[...]) + b_ref[...], 0.0)
    o_ref[...] = dinv * jnp.dot(a, w_ref[...],
                                preferred_element_type=f32, precision=_PREC)


def _tc_layer(s, u, dinv, b_row, W_next):
    return pl.pallas_call(
        _tc_layer_body,
        grid=(NPAD // BR,),
        in_specs=[
            pl.BlockSpec((2, BR, H), lambda i: (0, i, 0)),
            pl.BlockSpec((BR, H), lambda i: (i, 0)),
            pl.BlockSpec((BR, 1), lambda i: (i, 0)),
            pl.BlockSpec((1, H), lambda i: (0, 0)),
            pl.BlockSpec((H, H), lambda i: (0, 0)),
        ],
        out_specs=pl.BlockSpec((BR, H), lambda i: (i, 0)),
        out_shape=jax.ShapeDtypeStruct((NPAD, H), f32),
    )(s, u, dinv, b_row, W_next)


def _tc_pool_body(s_ref, u_ref, dinv_ref, b_ref, batch_ref, pooled_ref, cnt_ref):
    i = pl.program_id(0)

    @pl.when(i == 0)
    def _():
        pooled_ref[...] = jnp.zeros_like(pooled_ref)
        cnt_ref[...] = jnp.zeros_like(cnt_ref)

    dinv = dinv_ref[...]
    h = jnp.maximum(dinv * (s_ref[0] + s_ref[1] + u_ref[...]) + b_ref[...], 0.0)
    gids = lax.broadcasted_iota(jnp.int32, (G, BR), 0)
    oh = (batch_ref[...] == gids).astype(f32)
    pooled_ref[...] += jnp.dot(oh, h, preferred_element_type=f32,
                               precision=_PREC)
    cnt_ref[...] += jnp.dot(oh, jnp.ones((BR, H), f32),
                            preferred_element_type=f32, precision=_PREC)


def _tc_pool(s, u, dinv, b_row, batch_row):
    return pl.pallas_call(
        _tc_pool_body,
        grid=(NPAD // BR,),
        in_specs=[
            pl.BlockSpec((2, BR, H), lambda i: (0, i, 0)),
            pl.BlockSpec((BR, H), lambda i: (i, 0)),
            pl.BlockSpec((BR, 1), lambda i: (i, 0)),
            pl.BlockSpec((1, H), lambda i: (0, 0)),
            pl.BlockSpec((1, BR), lambda i: (0, i)),
        ],
        out_specs=[
            pl.BlockSpec((G, H), lambda i: (0, 0)),
            pl.BlockSpec((G, H), lambda i: (0, 0)),
        ],
        out_shape=[
            jax.ShapeDtypeStruct((G, H), f32),
            jax.ShapeDtypeStruct((G, H), f32),
        ],
    )(s, u, dinv, b_row, batch_row)


def _tc_head_body(pooled_ref, cnt_ref, pf_ref, wp1_ref, bp1_ref, wp2_ref,
                  bp2_ref, wc1a_ref, wc1b_ref, bc1_ref, wc2_ref, bc2_ref,
                  o_ref):
    drug = pooled_ref[...] / jnp.maximum(cnt_ref[...], 1.0)
    p = jnp.maximum(jnp.dot(pf_ref[...], wp1_ref[...],
                            preferred_element_type=f32, precision=_PREC)
                    + bp1_ref[...], 0.0)
    pe = jnp.maximum(jnp.dot(p, wp2_ref[...],
                             preferred_element_type=f32, precision=_PREC)
                     + bp2_ref[...], 0.0)
    z = jnp.maximum(jnp.dot(drug, wc1a_ref[...],
                            preferred_element_type=f32, precision=_PREC)
                    + jnp.dot(pe, wc1b_ref[...],
                              preferred_element_type=f32, precision=_PREC)
                    + bc1_ref[...], 0.0)
    o_ref[...] = jnp.dot(z, wc2_ref[...],
                         preferred_element_type=f32, precision=_PREC) + bc2_ref[...]


def _tc_head(pooled, cnt, pf_pad, Wp1p, bp1p, Wp2p, bp2p, Wc1a, Wc1b, bc1r,
             Wc2p, bc2p):
    return pl.pallas_call(
        _tc_head_body,
        out_shape=jax.ShapeDtypeStruct((G, OUTP), f32),
    )(pooled, cnt, pf_pad, Wp1p, bp1p, Wp2p, bp2p, Wc1a, Wc1b, bc1r, Wc2p,
      bc2p)


def kernel(x, edge_index, batch, patient_features, W1, b1, W2, b2, W3, b3,
           Wp1, bp1, Wp2, bp2, Wc1, bc1, Wc2, bc2):
    x_pad = jnp.zeros((NPAD, D), f32).at[:N].set(x)
    srcr = edge_index[0].reshape(NW, EPT_REAL)
    dstr = edge_index[1].reshape(NW, EPT_REAL)
    padcol = jnp.full((NW, EPT - EPT_REAL), SINK, jnp.int32)
    src_t = jnp.concatenate([srcr, padcol], axis=1).reshape(NW, CHUNKS, 128)
    dst_t = jnp.concatenate([dstr, padcol], axis=1).reshape(NW, CHUNKS, 128)

    ones128 = jnp.ones((128,), f32)
    zrow = jnp.zeros((RPT,), f32)
    zrows = jnp.zeros((RPT, H), f32)
    batch_row = jnp.full((1, NPAD), G, jnp.int32).at[0, :N].set(batch)

    b1r = b1[None, :]
    b2r = b2[None, :]
    b3r = b3[None, :]
    pf_pad = jnp.zeros((G, 128), f32).at[:, :PD].set(patient_features)
    Wp1p = jnp.zeros((128, 128), f32).at[:PD, :32].set(Wp1)
    bp1p = jnp.zeros((1, 128), f32).at[0, :32].set(bp1)
    Wp2p = jnp.zeros((128, 128), f32).at[:32, :PH].set(Wp2)
    bp2p = jnp.zeros((1, 128), f32).at[0, :PH].set(bp2)
    Wc1a = Wc1[:H]
    Wc1b = jnp.zeros((128, G), f32).at[:PH].set(Wc1[H:])
    bc1r = bc1[None, :]
    Wc2p = jnp.zeros((G, OUTP), f32).at[:, :OUT].set(Wc2)
    bc2p = jnp.zeros((1, OUTP), f32).at[0, :OUT].set(bc2)

    deg = _sc_degree(dst_t, ones128, zrow)
    u1, dinv = _tc_mm1(x_pad, W1, deg[0][:, None], deg[1][:, None])

    s1 = _sc_edge_agg(u1, src_t, dst_t, zrows)
    u2 = _tc_layer(s1, u1, dinv, b1r, W2)
    s2 = _sc_edge_agg(u2, src_t, dst_t, zrows)
    u3 = _tc_layer(s2, u2, dinv, b2r, W3)
    s3 = _sc_edge_agg(u3, src_t, dst_t, zrows)

    pooled, cnt = _tc_pool(s3, u3, dinv, b3r, batch_row)
    logits = _tc_head(pooled, cnt, pf_pad, Wp1p, bp1p, Wp2p, bp2p, Wc1a,
                      Wc1b, bc1r, Wc2p, bc2p)
    return logits[:, :OUT]


# speedup vs baseline: 8.8057x; 1.0424x over previous
import functools

import jax
import jax.numpy as jnp
from jax import lax
from jax.experimental import pallas as pl
from jax.experimental.pallas import tpu as pltpu
from jax.experimental.pallas import tpu_sc as plsc

N = 10000
E = 320000
D = 128
H = 128
G = 256
PD = 3
PH = 64
OUT = 1500

NC = 2
NS = 16
NW = NC * NS
NPAD = 10240
RPT = NPAD // NS
EPT_REAL = E // NW
CHUNKS = 80
EPT = CHUNKS * 128
GRP = 8
NGRP = CHUNKS // GRP
SINK = N
BR = 256
OUTP = 1536

f32 = jnp.float32
_PREC = lax.Precision.HIGHEST

_MESH = plsc.VectorSubcoreMesh(core_axis_name="c", subcore_axis_name="s")


@functools.partial(
    pl.kernel,
    out_type=jax.ShapeDtypeStruct((NC, NPAD), f32),
    mesh=_MESH,
    scratch_types=[
        pltpu.VMEM((CHUNKS, 128), jnp.int32),
        pltpu.VMEM((128,), f32),
        pltpu.VMEM_SHARED((NPAD,), f32),
    ],
)
def _sc_degree(dst_hbm, ones_hbm, zeros_hbm, out_hbm, dst_v, ones_v, acc):
    cid = lax.axis_index("c")
    sid = lax.axis_index("s")
    wid = sid * NC + cid
    pltpu.sync_copy(dst_hbm.at[wid], dst_v)
    pltpu.sync_copy(ones_hbm, ones_v)
    pltpu.sync_copy(zeros_hbm, acc.at[pl.ds(sid * RPT, RPT)])
    plsc.subcore_barrier()

    def body(c, carry):
        pltpu.sync_copy(ones_v, acc.at[dst_v.at[c]], add=True)
        return carry

    lax.fori_loop(0, CHUNKS, body, 0)
    plsc.subcore_barrier()
    pltpu.sync_copy(acc.at[pl.ds(sid * RPT, RPT)],
                    out_hbm.at[cid, pl.ds(sid * RPT, RPT)])


@functools.partial(
    pl.kernel,
    out_type=jax.ShapeDtypeStruct((NC, NPAD, H), f32),
    mesh=_MESH,
    scratch_types=[
        pltpu.VMEM((2, GRP, 128), jnp.int32),
        pltpu.VMEM((2, GRP, 128), jnp.int32),
        pltpu.VMEM((2, 128, H), f32),
        pltpu.VMEM_SHARED((NPAD, H), f32),
        pltpu.SemaphoreType.DMA,
        pltpu.SemaphoreType.DMA,
        pltpu.SemaphoreType.DMA,
    ],
)
def _sc_edge_agg(u_hbm, src_hbm, dst_hbm, zeros_hbm, out_hbm,
                 src_v, dst_v, rows_v, acc, sem0, sem1, semi):
    cid = lax.axis_index("c")
    sid = lax.axis_index("s")
    wid = sid * NC + cid
    pltpu.sync_copy(zeros_hbm, acc.at[pl.ds(sid * RPT, RPT)])
    plsc.subcore_barrier()

    def idx_fetch(g, gb):
        pltpu.async_copy(src_hbm.at[wid, pl.ds(g * GRP, GRP)], src_v.at[gb],
                         semi)
        pltpu.async_copy(dst_hbm.at[wid, pl.ds(g * GRP, GRP)], dst_v.at[gb],
                         semi)

    def idx_wait(gb):
        pltpu.make_async_copy(src_hbm.at[wid, pl.ds(0, GRP)], src_v.at[gb],
                              semi).wait()
        pltpu.make_async_copy(dst_hbm.at[wid, pl.ds(0, GRP)], dst_v.at[gb],
                              semi).wait()

    idx_fetch(0, 0)
    idx_wait(0)
    pltpu.async_copy(u_hbm.at[src_v.at[0, 0]], rows_v.at[0], sem0)

    for g in range(NGRP):
        gb = g % 2
        nxt = g + 1 < NGRP
        if nxt:
            idx_fetch(g + 1, 1 - gb)

        def pair(i, c2, gb=gb):
            ca = 2 * i
            cb = ca + 1
            db = pltpu.async_copy(u_hbm.at[src_v.at[gb, cb]], rows_v.at[1],
                                  sem1)
            pltpu.make_async_copy(u_hbm.at[src_v.at[gb, ca]], rows_v.at[0],
                                  sem0).wait()
            pltpu.sync_copy(rows_v.at[0], acc.at[dst_v.at[gb, ca]], add=True)
            pltpu.async_copy(u_hbm.at[src_v.at[gb, ca + 2]], rows_v.at[0],
                             sem0)
            db.wait()
            pltpu.sync_copy(rows_v.at[1], acc.at[dst_v.at[gb, cb]], add=True)
            return c2

        lax.fori_loop(0, GRP // 2 - 1, pair, 0)
        ca = GRP - 2
        cb = GRP - 1
        db = pltpu.async_copy(u_hbm.at[src_v.at[gb, cb]], rows_v.at[1], sem1)
        pltpu.make_async_copy(u_hbm.at[src_v.at[gb, ca]], rows_v.at[0],
                              sem0).wait()
        pltpu.sync_copy(rows_v.at[0], acc.at[dst_v.at[gb, ca]], add=True)
        if nxt:
            idx_wait(1 - gb)
            pltpu.async_copy(u_hbm.at[src_v.at[1 - gb, 0]], rows_v.at[0],
                             sem0)
        db.wait()
        pltpu.sync_copy(rows_v.at[1], acc.at[dst_v.at[gb, cb]], add=True)

    plsc.subcore_barrier()
    pltpu.sync_copy(acc.at[pl.ds(sid * RPT, RPT)],
                    out_hbm.at[cid, pl.ds(sid * RPT, RPT)])


def _tc_mm1_body(x_ref, w_ref, d0_ref, d1_ref, u_ref, dinv_ref):
    dinv = lax.rsqrt(1.0 + d0_ref[...] + d1_ref[...])
    dinv_ref[...] = dinv
    u_ref[...] = dinv * jnp.dot(x_ref[...], w_ref[...],
                                preferred_element_type=f32, precision=_PREC)


def _tc_mm1(x_pad, W1, d0, d1):
    return pl.pallas_call(
        _tc_mm1_body,
        grid=(NPAD // BR,),
        in_specs=[
            pl.BlockSpec((BR, D), lambda i: (i, 0)),
            pl.BlockSpec((D, H), lambda i: (0, 0)),
            pl.BlockSpec((BR, 1), lambda i: (i, 0)),
            pl.BlockSpec((BR, 1), lambda i: (i, 0)),
        ],
        out_specs=[
            pl.BlockSpec((BR, H), lambda i: (i, 0)),
            pl.BlockSpec((BR, 1), lambda i: (i, 0)),
        ],
        out_shape=[
            jax.ShapeDtypeStruct((NPAD, H), f32),
            jax.ShapeDtypeStruct((NPAD, 1), f32),
        ],
    )(x_pad, W1, d0, d1)


def _tc_layer_body(s_ref, u_ref, dinv_ref, b_ref, w_ref, o_ref):
    dinv = dinv_ref[...]
    a = jnp.maximum(dinv * (s_ref[0] + s_ref[1] + u_ref[...]) + b_ref[...], 0.0)
    o_ref[...] = dinv * jnp.dot(a, w_ref[...],
                                preferred_element_type=f32, precision=_PREC)


def _tc_layer(s, u, dinv, b_row, W_next):
    return pl.pallas_call(
        _tc_layer_body,
        grid=(NPAD // BR,),
        in_specs=[
            pl.BlockSpec((2, BR, H), lambda i: (0, i, 0)),
            pl.BlockSpec((BR, H), lambda i: (i, 0)),
            pl.BlockSpec((BR, 1), lambda i: (i, 0)),
            pl.BlockSpec((1, H), lambda i: (0, 0)),
            pl.BlockSpec((H, H), lambda i: (0, 0)),
        ],
        out_specs=pl.BlockSpec((BR, H), lambda i: (i, 0)),
        out_shape=jax.ShapeDtypeStruct((NPAD, H), f32),
    )(s, u, dinv, b_row, W_next)


def _tc_pool_body(s_ref, u_ref, dinv_ref, b_ref, batch_ref, pooled_ref, cnt_ref):
    i = pl.program_id(0)

    @pl.when(i == 0)
    def _():
        pooled_ref[...] = jnp.zeros_like(pooled_ref)
        cnt_ref[...] = jnp.zeros_like(cnt_ref)

    dinv = dinv_ref[...]
    h = jnp.maximum(dinv * (s_ref[0] + s_ref[1] + u_ref[...]) + b_ref[...], 0.0)
    gids = lax.broadcasted_iota(jnp.int32, (G, BR), 0)
    oh = (batch_ref[...] == gids).astype(f32)
    pooled_ref[...] += jnp.dot(oh, h, preferred_element_type=f32,
                               precision=_PREC)
    cnt_ref[...] += jnp.dot(oh, jnp.ones((BR, H), f32),
                            preferred_element_type=f32, precision=_PREC)


def _tc_pool(s, u, dinv, b_row, batch_row):
    return pl.pallas_call(
        _tc_pool_body,
        grid=(NPAD // BR,),
        in_specs=[
            pl.BlockSpec((2, BR, H), lambda i: (0, i, 0)),
            pl.BlockSpec((BR, H), lambda i: (i, 0)),
            pl.BlockSpec((BR, 1), lambda i: (i, 0)),
            pl.BlockSpec((1, H), lambda i: (0, 0)),
            pl.BlockSpec((1, BR), lambda i: (0, i)),
        ],
        out_specs=[
            pl.BlockSpec((G, H), lambda i: (0, 0)),
            pl.BlockSpec((G, H), lambda i: (0, 0)),
        ],
        out_shape=[
            jax.ShapeDtypeStruct((G, H), f32),
            jax.ShapeDtypeStruct((G, H), f32),
        ],
    )(s, u, dinv, b_row, batch_row)


def _tc_head_body(pooled_ref, cnt_ref, pf_ref, wp1_ref, bp1_ref, wp2_ref,
                  bp2_ref, wc1a_ref, wc1b_ref, bc1_ref, wc2_ref, bc2_ref,
                  o_ref):
    drug = pooled_ref[...] / jnp.maximum(cnt_ref[...], 1.0)
    p = jnp.maximum(jnp.dot(pf_ref[...], wp1_ref[...],
                            preferred_element_type=f32, precision=_PREC)
                    + bp1_ref[...], 0.0)
    pe = jnp.maximum(jnp.dot(p, wp2_ref[...],
                             preferred_element_type=f32, precision=_PREC)
                     + bp2_ref[...], 0.0)
    z = jnp.maximum(jnp.dot(drug, wc1a_ref[...],
                            preferred_element_type=f32, precision=_PREC)
                    + jnp.dot(pe, wc1b_ref[...],
                              preferred_element_type=f32, precision=_PREC)
                    + bc1_ref[...], 0.0)
    o_ref[...] = jnp.dot(z, wc2_ref[...],
                         preferred_element_type=f32, precision=_PREC) + bc2_ref[...]


def _tc_head(pooled, cnt, pf_pad, Wp1p, bp1p, Wp2p, bp2p, Wc1a, Wc1b, bc1r,
             Wc2p, bc2p):
    return pl.pallas_call(
        _tc_head_body,
        out_shape=jax.ShapeDtypeStruct((G, OUTP), f32),
    )(pooled, cnt, pf_pad, Wp1p, bp1p, Wp2p, bp2p, Wc1a, Wc1b, bc1r, Wc2p,
      bc2p)


def kernel(x, edge_index, batch, patient_features, W1, b1, W2, b2, W3, b3,
           Wp1, bp1, Wp2, bp2, Wc1, bc1, Wc2, bc2):
    x_pad = jnp.zeros((NPAD, D), f32).at[:N].set(x)
    srcr = edge_index[0].reshape(NW, EPT_REAL)
    dstr = edge_index[1].reshape(NW, EPT_REAL)
    padcol = jnp.full((NW, EPT - EPT_REAL), SINK, jnp.int32)
    src_t = jnp.concatenate([srcr, padcol], axis=1).reshape(NW, CHUNKS, 128)
    dst_t = jnp.concatenate([dstr, padcol], axis=1).reshape(NW, CHUNKS, 128)

    ones128 = jnp.ones((128,), f32)
    zrow = jnp.zeros((RPT,), f32)
    zrows = jnp.zeros((RPT, H), f32)
    batch_row = jnp.full((1, NPAD), G, jnp.int32).at[0, :N].set(batch)

    b1r = b1[None, :]
    b2r = b2[None, :]
    b3r = b3[None, :]
    pf_pad = jnp.zeros((G, 128), f32).at[:, :PD].set(patient_features)
    Wp1p = jnp.zeros((128, 128), f32).at[:PD, :32].set(Wp1)
    bp1p = jnp.zeros((1, 128), f32).at[0, :32].set(bp1)
    Wp2p = jnp.zeros((128, 128), f32).at[:32, :PH].set(Wp2)
    bp2p = jnp.zeros((1, 128), f32).at[0, :PH].set(bp2)
    Wc1a = Wc1[:H]
    Wc1b = jnp.zeros((128, G), f32).at[:PH].set(Wc1[H:])
    bc1r = bc1[None, :]
    Wc2p = jnp.zeros((G, OUTP), f32).at[:, :OUT].set(Wc2)
    bc2p = jnp.zeros((1, OUTP), f32).at[0, :OUT].set(bc2)

    deg = _sc_degree(dst_t, ones128, zrow)
    u1, dinv = _tc_mm1(x_pad, W1, deg[0][:, None], deg[1][:, None])

    s1 = _sc_edge_agg(u1, src_t, dst_t, zrows)
    u2 = _tc_layer(s1, u1, dinv, b1r, W2)
    s2 = _sc_edge_agg(u2, src_t, dst_t, zrows)
    u3 = _tc_layer(s2, u2, dinv, b2r, W3)
    s3 = _sc_edge_agg(u3, src_t, dst_t, zrows)

    pooled, cnt = _tc_pool(s3, u3, dinv, b3r, batch_row)
    logits = _tc_head(pooled, cnt, pf_pad, Wp1p, bp1p, Wp2p, bp2p, Wc1a,
                      Wc1b, bc1r, Wc2p, bc2p)
    return logits[:, :OUT]
